# Initial kernel scaffold; baseline (speedup 1.0000x reference)
import jax
import jax.numpy as jnp
from jax.experimental import pallas as pl


def kernel(points, view_dirs, density_grid, sh_grid):
    raise NotImplementedError("write your pallas kernel here")


# speedup vs baseline: 7.1146x; 7.1146x over previous
import functools

import jax
import jax.numpy as jnp
from jax import lax
from jax.experimental import pallas as pl
from jax.experimental.pallas import tpu as pltpu
from jax.experimental.pallas import tpu_sc as plsc

N_PTS = 262144
RES = 128
NROWS = RES * RES * RES
C_SH = 27
D_ROW = 32
L = 16

OFFS = (0, 1, 128, 129, 16384, 16385, 16512, 16513)


def _splat_i32(v):
    return jnp.full((L,), v, dtype=jnp.int32)


def _rsqrt(s):
    i = lax.bitcast_convert_type(s, jnp.int32)
    y = lax.bitcast_convert_type(
        0x5F3759DF - lax.shift_right_logical(i, 1), jnp.float32
    )
    for _ in range(3):
        y = y * (1.5 - 0.5 * s * y * y)
    return y


def _sigmoid(x):
    return 1.0 / (1.0 + jnp.exp(-x))


def _make_kernel(n_per_w, nc, ns):
    n_chunks = n_per_w // L
    mesh = plsc.VectorSubcoreMesh(core_axis_name="c", subcore_axis_name="s")

    @functools.partial(
        pl.kernel,
        mesh=mesh,
        out_type=[
            jax.ShapeDtypeStruct((N_PTS,), jnp.float32),
            jax.ShapeDtypeStruct((N_PTS, 3), jnp.float32),
        ],
        scratch_types=[
            pltpu.VMEM((3, n_per_w), jnp.float32),
            pltpu.VMEM((3, n_per_w), jnp.float32),
            pltpu.VMEM((8 * L,), jnp.int32),
            pltpu.VMEM((8 * L, D_ROW), jnp.float32),
            pltpu.VMEM((n_per_w,), jnp.float32),
            pltpu.VMEM((n_per_w, 3), jnp.float32),
            pltpu.SemaphoreType.DMA,
        ],
        compiler_params=pltpu.CompilerParams(
            needs_layout_passes=False, use_tc_tiling_on_sc=False
        ),
    )
    def k(tab_hbm, pts_hbm, dirs_hbm, out_d, out_c,
          pts_v, dirs_v, idx_v, rows_v, dens_v, cols_v, sem):
        wid = lax.axis_index("s") * nc + lax.axis_index("c")
        base_pt = wid * n_per_w

        pltpu.sync_copy(pts_hbm.at[wid], pts_v)
        pltpu.sync_copy(dirs_hbm.at[wid], dirs_v)

        iota = lax.iota(jnp.int32, L)

        def chunk(g, carry):
            o = pl.multiple_of(g * L, L)
            px = pts_v[0, pl.ds(o, L)]
            py = pts_v[1, pl.ds(o, L)]
            pz = pts_v[2, pl.ds(o, L)]

            def grid_coord(p):
                t = jnp.minimum(jnp.maximum(p * (2.0 / 3.0), -1.0), 1.0)
                gc = (t + 1.0) * 63.5
                gi0 = gc.astype(jnp.int32)
                gi0 = jnp.where(gi0.astype(jnp.float32) > gc, gi0 - 1, gi0)
                gi = jnp.minimum(jnp.maximum(gi0, 0), 126)
                return gi, gc - gi.astype(jnp.float32)

            xi, fx = grid_coord(px)
            yi, fy = grid_coord(py)
            zi, fz = grid_coord(pz)
            vbase = xi * 16384 + yi * 128 + zi

            for c in range(8):
                idx_v[pl.ds(c * L, L)] = vbase + OFFS[c]

            cp = pltpu.async_copy(tab_hbm.at[idx_v], rows_v, sem)

            ex, ey, ez = 1.0 - fx, 1.0 - fy, 1.0 - fz
            w = [None] * 8
            for dx in (0, 1):
                for dy in (0, 1):
                    wxy = (fx if dx else ex) * (fy if dy else ey)
                    w[dx * 4 + dy * 2] = wxy * ez
                    w[dx * 4 + dy * 2 + 1] = wxy * fz

            dxv = dirs_v[0, pl.ds(o, L)]
            dyv = dirs_v[1, pl.ds(o, L)]
            dzv = dirs_v[2, pl.ds(o, L)]
            sq = dxv * dxv + dyv * dyv + dzv * dzv
            small = sq < 1e-8
            dxv = jnp.where(small, 0.0, dxv)
            dyv = jnp.where(small, 0.0, dyv)
            dzv = jnp.where(small, 1.0, dzv)
            s2 = jnp.where(small, 1.0, sq)
            r = _rsqrt(s2)
            x = dxv * r
            yb = dyv * r
            z = dzv * r
            basis = [
                0.282095 * jnp.ones_like(x),
                0.488603 * yb,
                0.488603 * z,
                0.488603 * x,
                1.092548 * x * yb,
                1.092548 * yb * z,
                0.315392 * (3.0 * z * z - 1.0),
                1.092548 * x * z,
                0.546274 * (x * x - yb * yb),
            ]

            cp.wait()

            col = [jnp.zeros((L,), jnp.float32) for _ in range(3)]
            dens = jnp.zeros((L,), jnp.float32)
            for c in range(8):
                ridx = iota + c * L
                for j in range(C_SH):
                    v = plsc.load_gather(rows_v, [ridx, _splat_i32(j)])
                    ch, kk = j // 9, j % 9
                    col[ch] = col[ch] + basis[kk] * (w[c] * v)
                dv = plsc.load_gather(rows_v, [ridx, _splat_i32(27)])
                dens = dens + w[c] * dv

            dens_v[pl.ds(o, L)] = jnp.maximum(dens, 0.0)
            rowp = iota + o
            for ch in range(3):
                plsc.store_scatter(
                    cols_v, [rowp, _splat_i32(ch)], _sigmoid(col[ch])
                )
            return carry

        lax.fori_loop(0, n_chunks, chunk, 0)

        pltpu.sync_copy(dens_v, out_d.at[pl.ds(base_pt, n_per_w)])
        pltpu.sync_copy(cols_v, out_c.at[pl.ds(base_pt, n_per_w), :])

    return k


def kernel(points, view_dirs, density_grid, sh_grid):
    info = plsc.get_sparse_core_info()
    nc, ns = info.num_cores, info.num_subcores
    nw = nc * ns
    n_per_w = N_PTS // nw

    tab = jnp.concatenate(
        [
            sh_grid.reshape(NROWS, C_SH),
            density_grid.reshape(NROWS, 1),
            jnp.zeros((NROWS, D_ROW - C_SH - 1), jnp.float32),
        ],
        axis=1,
    )
    pts_r = points.T.reshape(3, nw, n_per_w).transpose(1, 0, 2)
    dirs_r = view_dirs.T.reshape(3, nw, n_per_w).transpose(1, 0, 2)

    k = _make_kernel(n_per_w, nc, ns)
    density, colors = k(tab, pts_r, dirs_r)
    return density, colors


# speedup vs baseline: 7.5130x; 1.0560x over previous
import functools

import jax
import jax.numpy as jnp
from jax import lax
from jax.experimental import pallas as pl
from jax.experimental.pallas import tpu as pltpu
from jax.experimental.pallas import tpu_sc as plsc

N_PTS = 262144
RES = 128
NROWS = RES * RES * RES
C_SH = 27
D_ROW = 32
L = 16

OFFS = (0, 1, 128, 129, 16384, 16385, 16512, 16513)


def _splat_i32(v):
    return jnp.full((L,), v, dtype=jnp.int32)


def _rsqrt(s):
    i = lax.bitcast_convert_type(s, jnp.int32)
    y = lax.bitcast_convert_type(
        0x5F3759DF - lax.shift_right_logical(i, 1), jnp.float32
    )
    for _ in range(3):
        y = y * (1.5 - 0.5 * s * y * y)
    return y


def _sigmoid(x):
    return 1.0 / (1.0 + jnp.exp(-x))


def _make_kernel(n_per_w, nc, ns):
    n_chunks = n_per_w // L
    mesh = plsc.VectorSubcoreMesh(core_axis_name="c", subcore_axis_name="s")

    @functools.partial(
        pl.kernel,
        mesh=mesh,
        out_type=[
            jax.ShapeDtypeStruct((N_PTS,), jnp.float32),
            jax.ShapeDtypeStruct((3, N_PTS), jnp.float32),
        ],
        scratch_types=[
            pltpu.VMEM((3, n_per_w), jnp.float32),
            pltpu.VMEM((3, n_per_w), jnp.float32),
            pltpu.VMEM((8 * L,), jnp.int32),
            pltpu.VMEM((8 * L,), jnp.int32),
            pltpu.VMEM((8 * L, D_ROW), jnp.float32),
            pltpu.VMEM((8 * L, D_ROW), jnp.float32),
            pltpu.VMEM((n_per_w,), jnp.float32),
            pltpu.VMEM((3, n_per_w), jnp.float32),
            pltpu.SemaphoreType.DMA,
            pltpu.SemaphoreType.DMA,
        ],
        compiler_params=pltpu.CompilerParams(
            needs_layout_passes=False, use_tc_tiling_on_sc=False
        ),
    )
    def k(tab_hbm, pts_hbm, dirs_hbm, out_d, out_c,
          pts_v, dirs_v, idx0, idx1, rows0, rows1, dens_v, cols_v,
          sem0, sem1):
        wid = lax.axis_index("s") * nc + lax.axis_index("c")
        base_pt = wid * n_per_w

        pltpu.sync_copy(pts_hbm.at[wid], pts_v)
        pltpu.sync_copy(dirs_hbm.at[wid], dirs_v)

        iota = lax.iota(jnp.int32, L)

        def grid_coord(p):
            t = jnp.minimum(jnp.maximum(p * (2.0 / 3.0), -1.0), 1.0)
            gc = (t + 1.0) * 63.5
            gi0 = gc.astype(jnp.int32)
            gi0 = jnp.where(gi0.astype(jnp.float32) > gc, gi0 - 1, gi0)
            gi = jnp.minimum(jnp.maximum(gi0, 0), 126)
            return gi, gc - gi.astype(jnp.float32)

        def load_pt(g):
            o = pl.multiple_of(g * L, L)
            return (pts_v[0, pl.ds(o, L)], pts_v[1, pl.ds(o, L)],
                    pts_v[2, pl.ds(o, L)])

        def fire(g, idx_r, rows_r, sem_r):
            px, py, pz = load_pt(g)
            xi, _ = grid_coord(px)
            yi, _ = grid_coord(py)
            zi, _ = grid_coord(pz)
            vbase = xi * 16384 + yi * 128 + zi
            for c in range(8):
                idx_r[pl.ds(c * L, L)] = vbase + OFFS[c]
            pltpu.async_copy(tab_hbm.at[idx_r], rows_r, sem_r)

        def drain(idx_r, rows_r, sem_r):
            pltpu.make_async_copy(tab_hbm.at[idx_r], rows_r, sem_r).wait()

        def compute(g, rows_r):
            o = pl.multiple_of(g * L, L)
            px, py, pz = load_pt(g)
            _, fx = grid_coord(px)
            _, fy = grid_coord(py)
            _, fz = grid_coord(pz)

            ex, ey, ez = 1.0 - fx, 1.0 - fy, 1.0 - fz
            w = [None] * 8
            for dx in (0, 1):
                for dy in (0, 1):
                    wxy = (fx if dx else ex) * (fy if dy else ey)
                    w[dx * 4 + dy * 2] = wxy * ez
                    w[dx * 4 + dy * 2 + 1] = wxy * fz

            dxv = dirs_v[0, pl.ds(o, L)]
            dyv = dirs_v[1, pl.ds(o, L)]
            dzv = dirs_v[2, pl.ds(o, L)]
            sq = dxv * dxv + dyv * dyv + dzv * dzv
            small = sq < 1e-8
            dxv = jnp.where(small, 0.0, dxv)
            dyv = jnp.where(small, 0.0, dyv)
            dzv = jnp.where(small, 1.0, dzv)
            s2 = jnp.where(small, 1.0, sq)
            r = _rsqrt(s2)
            x = dxv * r
            yb = dyv * r
            z = dzv * r
            basis = [
                0.282095 * jnp.ones_like(x),
                0.488603 * yb,
                0.488603 * z,
                0.488603 * x,
                1.092548 * x * yb,
                1.092548 * yb * z,
                0.315392 * (3.0 * z * z - 1.0),
                1.092548 * x * z,
                0.546274 * (x * x - yb * yb),
            ]

            col = [jnp.zeros((L,), jnp.float32) for _ in range(3)]
            dens = jnp.zeros((L,), jnp.float32)
            for c in range(8):
                ridx = iota + c * L
                for j in range(C_SH):
                    v = plsc.load_gather(rows_r, [ridx, _splat_i32(j)])
                    ch, kk = j // 9, j % 9
                    col[ch] = col[ch] + basis[kk] * (w[c] * v)
                dv = plsc.load_gather(rows_r, [ridx, _splat_i32(27)])
                dens = dens + w[c] * dv

            dens_v[pl.ds(o, L)] = jnp.maximum(dens, 0.0)
            for ch in range(3):
                cols_v[ch, pl.ds(o, L)] = _sigmoid(col[ch])

        fire(0, idx0, rows0, sem0)

        def body(i, carry):
            g0 = i * 2
            fire(g0 + 1, idx1, rows1, sem1)
            drain(idx0, rows0, sem0)
            compute(g0, rows0)

            @pl.when(g0 + 2 < n_chunks)
            def _():
                fire(g0 + 2, idx0, rows0, sem0)

            drain(idx1, rows1, sem1)
            compute(g0 + 1, rows1)
            return carry

        lax.fori_loop(0, n_chunks // 2, body, 0)

        pltpu.sync_copy(dens_v, out_d.at[pl.ds(base_pt, n_per_w)])
        for ch in range(3):
            pltpu.sync_copy(cols_v.at[ch], out_c.at[ch, pl.ds(base_pt, n_per_w)])

    return k


def kernel(points, view_dirs, density_grid, sh_grid):
    info = plsc.get_sparse_core_info()
    nc, ns = info.num_cores, info.num_subcores
    nw = nc * ns
    n_per_w = N_PTS // nw

    tab = jnp.concatenate(
        [
            sh_grid.reshape(NROWS, C_SH),
            density_grid.reshape(NROWS, 1),
            jnp.zeros((NROWS, D_ROW - C_SH - 1), jnp.float32),
        ],
        axis=1,
    )
    pts_r = points.T.reshape(3, nw, n_per_w).transpose(1, 0, 2)
    dirs_r = view_dirs.T.reshape(3, nw, n_per_w).transpose(1, 0, 2)

    k = _make_kernel(n_per_w, nc, ns)
    density, colors_t = k(tab, pts_r, dirs_r)
    return density, colors_t.T


# speedup vs baseline: 8.9067x; 1.1855x over previous
import functools

import jax
import jax.numpy as jnp
from jax import lax
from jax.experimental import pallas as pl
from jax.experimental.pallas import tpu as pltpu
from jax.experimental.pallas import tpu_sc as plsc

N_PTS = 262144
RES = 128
NROWS = RES * RES * RES
C_SH = 27
D_ROW = 32
L = 16

OFFS = (0, 1, 128, 129, 16384, 16385, 16512, 16513)


def _splat_i32(v):
    return jnp.full((L,), v, dtype=jnp.int32)


def _rsqrt(s):
    i = lax.bitcast_convert_type(s, jnp.int32)
    y = lax.bitcast_convert_type(
        0x5F3759DF - lax.shift_right_logical(i, 1), jnp.float32
    )
    for _ in range(3):
        y = y * (1.5 - 0.5 * s * y * y)
    return y


def _sigmoid(x):
    return 1.0 / (1.0 + jnp.exp(-x))


def _make_kernel(n_per_w, nc, ns):
    n_chunks = n_per_w // L
    mesh = plsc.VectorSubcoreMesh(core_axis_name="c", subcore_axis_name="s")

    @functools.partial(
        pl.kernel,
        mesh=mesh,
        out_type=[
            jax.ShapeDtypeStruct((N_PTS,), jnp.float32),
            jax.ShapeDtypeStruct((3, N_PTS), jnp.float32),
        ],
        scratch_types=[
            pltpu.VMEM((3, n_per_w), jnp.float32),
            pltpu.VMEM((3, n_per_w), jnp.float32),
            pltpu.VMEM((8 * L,), jnp.int32),
            pltpu.VMEM((8 * L,), jnp.int32),
            pltpu.VMEM((8 * L, 3, 16), jnp.float32),
            pltpu.VMEM((8 * L, 3, 16), jnp.float32),
            pltpu.VMEM((8 * L,), jnp.float32),
            pltpu.VMEM((8 * L,), jnp.float32),
            pltpu.VMEM((n_per_w,), jnp.float32),
            pltpu.VMEM((3, n_per_w), jnp.float32),
            pltpu.SemaphoreType.DMA,
            pltpu.SemaphoreType.DMA,
            pltpu.SemaphoreType.DMA,
            pltpu.SemaphoreType.DMA,
        ],
        compiler_params=pltpu.CompilerParams(
            needs_layout_passes=False, use_tc_tiling_on_sc=False
        ),
    )
    def k(tab_hbm, dflat_hbm, pts_hbm, dirs_hbm, out_d, out_c,
          pts_v, dirs_v, idx0, idx1, rows0, rows1, drow0, drow1,
          dens_v, cols_v, sem0, sem1, semd0, semd1):
        wid = lax.axis_index("s") * nc + lax.axis_index("c")
        base_pt = wid * n_per_w

        pltpu.sync_copy(pts_hbm.at[wid], pts_v)
        pltpu.sync_copy(dirs_hbm.at[wid], dirs_v)

        iota = lax.iota(jnp.int32, L)

        def grid_coord(p):
            t = jnp.minimum(jnp.maximum(p * (2.0 / 3.0), -1.0), 1.0)
            gc = (t + 1.0) * 63.5
            gi0 = gc.astype(jnp.int32)
            gi0 = jnp.where(gi0.astype(jnp.float32) > gc, gi0 - 1, gi0)
            gi = jnp.minimum(jnp.maximum(gi0, 0), 126)
            return gi, gc - gi.astype(jnp.float32)

        def load_pt(g):
            o = pl.multiple_of(g * L, L)
            return (pts_v[0, pl.ds(o, L)], pts_v[1, pl.ds(o, L)],
                    pts_v[2, pl.ds(o, L)])

        def fire(g, idx_r, rows_r, sem_r, drow_r, semd_r):
            px, py, pz = load_pt(g)
            xi, _ = grid_coord(px)
            yi, _ = grid_coord(py)
            zi, _ = grid_coord(pz)
            vbase = xi * 16384 + yi * 128 + zi
            for c in range(8):
                idx_r[pl.ds(c * L, L)] = vbase + OFFS[c]
            pltpu.async_copy(tab_hbm.at[idx_r], rows_r, sem_r)
            pltpu.async_copy(dflat_hbm.at[idx_r], drow_r, semd_r)

        def drain(idx_r, rows_r, sem_r, drow_r, semd_r):
            pltpu.make_async_copy(tab_hbm.at[idx_r], rows_r, sem_r).wait()
            pltpu.make_async_copy(dflat_hbm.at[idx_r], drow_r, semd_r).wait()

        def compute(g, rows_r, drow_r):
            o = pl.multiple_of(g * L, L)
            px, py, pz = load_pt(g)
            _, fx = grid_coord(px)
            _, fy = grid_coord(py)
            _, fz = grid_coord(pz)

            ex, ey, ez = 1.0 - fx, 1.0 - fy, 1.0 - fz
            w = [None] * 8
            for dx in (0, 1):
                for dy in (0, 1):
                    wxy = (fx if dx else ex) * (fy if dy else ey)
                    w[dx * 4 + dy * 2] = wxy * ez
                    w[dx * 4 + dy * 2 + 1] = wxy * fz

            dxv = dirs_v[0, pl.ds(o, L)]
            dyv = dirs_v[1, pl.ds(o, L)]
            dzv = dirs_v[2, pl.ds(o, L)]
            sq = dxv * dxv + dyv * dyv + dzv * dzv
            small = sq < 1e-8
            dxv = jnp.where(small, 0.0, dxv)
            dyv = jnp.where(small, 0.0, dyv)
            dzv = jnp.where(small, 1.0, dzv)
            s2 = jnp.where(small, 1.0, sq)
            r = _rsqrt(s2)
            x = dxv * r
            yb = dyv * r
            z = dzv * r
            basis = [
                0.282095 * jnp.ones_like(x),
                0.488603 * yb,
                0.488603 * z,
                0.488603 * x,
                1.092548 * x * yb,
                1.092548 * yb * z,
                0.315392 * (3.0 * z * z - 1.0),
                1.092548 * x * z,
                0.546274 * (x * x - yb * yb),
            ]

            col = [jnp.zeros((L,), jnp.float32) for _ in range(3)]
            dens = jnp.zeros((L,), jnp.float32)
            for c in range(8):
                ridx = iota + c * L
                for ch in range(3):
                    for kk in range(9):
                        v = plsc.load_gather(
                            rows_r, [ridx, _splat_i32(ch), _splat_i32(kk)]
                        )
                        col[ch] = col[ch] + basis[kk] * (w[c] * v)
                dens = dens + w[c] * drow_r[pl.ds(c * L, L)]

            dens_v[pl.ds(o, L)] = jnp.maximum(dens, 0.0)
            for ch in range(3):
                cols_v[ch, pl.ds(o, L)] = _sigmoid(col[ch])

        fire(0, idx0, rows0, sem0, drow0, semd0)

        def body(i, carry):
            g0 = i * 2
            fire(g0 + 1, idx1, rows1, sem1, drow1, semd1)
            drain(idx0, rows0, sem0, drow0, semd0)
            compute(g0, rows0, drow0)

            @pl.when(g0 + 2 < n_chunks)
            def _():
                fire(g0 + 2, idx0, rows0, sem0, drow0, semd0)

            drain(idx1, rows1, sem1, drow1, semd1)
            compute(g0 + 1, rows1, drow1)
            return carry

        lax.fori_loop(0, n_chunks // 2, body, 0)

        pltpu.sync_copy(dens_v, out_d.at[pl.ds(base_pt, n_per_w)])
        for ch in range(3):
            pltpu.sync_copy(cols_v.at[ch], out_c.at[ch, pl.ds(base_pt, n_per_w)])

    return k


def kernel(points, view_dirs, density_grid, sh_grid):
    info = plsc.get_sparse_core_info()
    nc, ns = info.num_cores, info.num_subcores
    nw = nc * ns
    n_per_w = N_PTS // nw

    tab = jnp.concatenate(
        [sh_grid.reshape(NROWS, 3, 9), jnp.zeros((NROWS, 3, 7), jnp.float32)],
        axis=2,
    )
    dflat = density_grid.reshape(NROWS)
    pts_r = points.T.reshape(3, nw, n_per_w).transpose(1, 0, 2)
    dirs_r = view_dirs.T.reshape(3, nw, n_per_w).transpose(1, 0, 2)

    k = _make_kernel(n_per_w, nc, ns)
    density, colors_t = k(tab, dflat, pts_r, dirs_r)
    return density, colors_t.T
